# baseline (speedup 1.0000x reference)
import functools

import jax
import jax.numpy as jnp
import numpy as np
from jax import lax
from jax.experimental import pallas as pl
from jax.experimental.pallas import tpu as pltpu
from jax.experimental.pallas import tpu_sc as plsc

N_ANCHORS = 20000
LANES = 128
ROWS = 160
NPAD = ROWS * LANES
PRE_NMS = 6000
N_OUT = 1000
IOU_THR = 0.7
NEG = np.float32(-1e38)

SC_TILES = 16
SC_PART = NPAD // SC_TILES
SC_CHUNKS = SC_PART // 128
COMP = 6144
CROWS = COMP // LANES
TRASH = 2 * COMP
COMP_ROWS = TRASH + 8
ROWW = 16


def _threshold_kernel(scores_ref, out_ref, outi_ref):
    scores = scores_ref[0]
    bits = lax.bitcast_convert_type(scores, jnp.int32)

    def count_ge(v):
        return jnp.sum((bits >= v).astype(jnp.int32))

    def bs_body(_, state):
        lo, hi = state
        mid = lo + (hi - lo) // 2
        ge = count_ge(mid) >= PRE_NMS
        return (jnp.where(ge, mid, lo), jnp.where(ge, hi, mid))

    lo, _ = lax.fori_loop(
        0, 31, bs_body, (jnp.int32(0), jnp.int32(np.int32(0x7F800000))))
    vstar = lo
    count_gt = jnp.sum((bits > vstar).astype(jnp.int32))
    k_ties = PRE_NMS - count_gt

    row_iota = lax.broadcasted_iota(jnp.int32, (ROWS, LANES), 0)
    col_iota = lax.broadcasted_iota(jnp.int32, (ROWS, LANES), 1)
    idx2d = row_iota * LANES + col_iota
    is_tie = bits == vstar

    def count_tie_lt(i):
        return jnp.sum((is_tie & (idx2d < i)).astype(jnp.int32))

    def bs2_body(_, state):
        lo2, hi2 = state
        mid = lo2 + (hi2 - lo2) // 2
        ge = count_tie_lt(mid) >= k_ties
        return (jnp.where(ge, lo2, mid), jnp.where(ge, mid, hi2))

    _, hi2 = lax.fori_loop(0, 15, bs2_body, (jnp.int32(0), jnp.int32(NPAD)))
    istar = hi2

    lane = lax.broadcasted_iota(jnp.int32, (1, LANES), 1)
    vstar_f = lax.bitcast_convert_type(vstar, jnp.float32)
    out_ref[0] = jnp.where(lane == 0, vstar_f, np.float32(0.0))
    outi_ref[0] = jnp.where(lane == 0, istar, 0)


def _compact_kernel(scores_hbm, geom_hbm, thrf_hbm, thri_hbm, comp_hbm,
                    sco_v, geo_v, thrf_v, thri_v, rows_v, idx_v, cnt_v,
                    counts_sh, allcnt_v, sem):
    b = lax.axis_index("c")
    part = lax.axis_index("s")
    base = part * SC_PART

    pltpu.sync_copy(scores_hbm.at[b, pl.ds(base, SC_PART)], sco_v)
    for ch in range(8):
        pltpu.sync_copy(geom_hbm.at[b, ch, pl.ds(base, SC_PART)],
                        geo_v.at[ch])
    pltpu.sync_copy(thrf_hbm.at[b], thrf_v)
    pltpu.sync_copy(thri_hbm.at[b], thri_v)
    vstar = thrf_v[pl.ds(0, 16)][0]
    istar = thri_v[pl.ds(0, 16)][0]
    lane = lax.broadcasted_iota(jnp.int32, (16,), 0)

    def active_mask(off):
        s16 = sco_v[pl.ds(off, 16)]
        gidx = base + off + lane
        return s16, (s16 > vstar) | ((s16 == vstar) & (gidx < istar))

    acc = jnp.zeros((16,), jnp.float32)
    for i in range(SC_PART // 16):
        _, m = active_mask(i * 16)
        acc = acc + jnp.where(m, jnp.float32(1), jnp.float32(0))
    cnt_v[...] = jnp.full((16,), jnp.sum(acc).astype(jnp.int32), jnp.int32)
    pltpu.sync_copy(cnt_v, counts_sh.at[part])
    plsc.subcore_barrier()
    pltpu.sync_copy(counts_sh, allcnt_v)

    goff = b * COMP
    for p in range(SC_TILES):
        cvec = allcnt_v[p, pl.ds(0, 16)]
        goff = goff + jnp.where(p < part, cvec[0], 0)

    one = np.float32(1.0)
    zero = np.float32(0.0)
    half = np.float32(0.5)
    handles = []
    cc = jnp.int32(0)
    for j in range(SC_CHUNKS):
        for kk in range(8):
            off = j * 128 + kk * 16
            s16, m = active_mask(off)
            sl = pl.ds(off, 16)
            ay1 = geo_v[0, sl]
            ax1 = geo_v[1, sl]
            ay2 = geo_v[2, sl]
            ax2 = geo_v[3, sl]
            dy = geo_v[4, sl] * np.float32(0.1)
            dx = geo_v[5, sl] * np.float32(0.1)
            dh = geo_v[6, sl] * np.float32(0.2)
            dw = geo_v[7, sl] * np.float32(0.2)
            height = ay2 - ay1
            width = ax2 - ax1
            center_y = ay1 + half * height
            center_x = ax1 + half * width
            center_y = center_y + dy * height
            center_x = center_x + dx * width
            height = height * jnp.exp(dh)
            width = width * jnp.exp(dw)
            y1 = center_y - half * height
            x1 = center_x - half * width
            y2 = y1 + height
            x2 = x1 + width
            y1 = jnp.maximum(jnp.minimum(y1, one), zero)
            x1 = jnp.maximum(jnp.minimum(x1, one), zero)
            y2 = jnp.maximum(jnp.minimum(y2, one), zero)
            x2 = jnp.maximum(jnp.minimum(x2, one), zero)
            area = (y2 - y1) * (x2 - x1)

            mf = jnp.where(m, jnp.float32(1), jnp.float32(0))
            excl = (plsc.cumsum(mf) - mf).astype(jnp.int32)
            rowidx = off + lane
            vals = (s16, y1, x1, y2, x2, area)
            for ch, v in enumerate(vals):
                plsc.store_scatter(
                    rows_v, [rowidx, jnp.full((16,), ch, jnp.int32)], v)
            idx_v[j, pl.ds(kk * 16, 16)] = jnp.where(
                m, goff + cc + excl, jnp.int32(TRASH))
            cc = cc + jnp.sum(mf).astype(jnp.int32)
        handles.append(pltpu.async_copy(
            rows_v.at[pl.ds(j * 128, 128)], comp_hbm.at[idx_v.at[j]], sem))
    for h in handles:
        h.wait()


_compact = functools.partial(
    pl.kernel,
    out_type=jax.ShapeDtypeStruct((COMP_ROWS, ROWW), jnp.float32),
    mesh=plsc.VectorSubcoreMesh(core_axis_name="c", subcore_axis_name="s"),
    compiler_params=pltpu.CompilerParams(
        needs_layout_passes=False, use_tc_tiling_on_sc=False),
    scratch_types=[
        pltpu.VMEM((SC_PART,), jnp.float32),
        pltpu.VMEM((8, SC_PART), jnp.float32),
        pltpu.VMEM((LANES,), jnp.float32),
        pltpu.VMEM((LANES,), jnp.int32),
        pltpu.VMEM((SC_PART, ROWW), jnp.float32),
        pltpu.VMEM((SC_CHUNKS, 128), jnp.int32),
        pltpu.VMEM((16,), jnp.int32),
        pltpu.VMEM_SHARED((SC_TILES, 16), jnp.int32),
        pltpu.VMEM((SC_TILES, 16), jnp.int32),
        pltpu.SemaphoreType.DMA,
    ],
)(_compact_kernel)


def _nms_kernel(comp_ref, out_ref, masked_ref):
    row_iota = lax.broadcasted_iota(jnp.int32, (CROWS, LANES), 0)
    col_iota = lax.broadcasted_iota(jnp.int32, (CROWS, LANES), 1)
    idx2d = row_iota * LANES + col_iota
    zero = np.float32(0.0)
    masked_ref[...] = jnp.where(idx2d < PRE_NMS, comp_ref[0, 0], NEG)

    out_ref[...] = jnp.zeros_like(out_ref)
    out_iota = (lax.broadcasted_iota(jnp.int32, (8, LANES), 0) * LANES
                + lax.broadcasted_iota(jnp.int32, (8, LANES), 1))
    thr = np.float32(IOU_THR)
    big = jnp.int32(2 ** 30)

    def nms_body(i, carry):
        masked = masked_ref[...]
        m = jnp.max(masked)

        @pl.when(m >= zero)
        def _():
            sel = masked == m
            j = jnp.min(jnp.where(sel, idx2d, big))
            selj = idx2d == j
            cy1 = comp_ref[0, 1]
            cx1 = comp_ref[0, 2]
            cy2 = comp_ref[0, 3]
            cx2 = comp_ref[0, 4]
            car = comp_ref[0, 5]
            fz = jnp.float32(0.0)
            by1 = jnp.sum(jnp.where(selj, cy1, fz))
            bx1 = jnp.sum(jnp.where(selj, cx1, fz))
            by2 = jnp.sum(jnp.where(selj, cy2, fz))
            bx2 = jnp.sum(jnp.where(selj, cx2, fz))
            barea = jnp.sum(jnp.where(selj, car, fz))
            yy1 = jnp.maximum(by1, cy1)
            xx1 = jnp.maximum(bx1, cx1)
            yy2 = jnp.minimum(by2, cy2)
            xx2 = jnp.minimum(bx2, cx2)
            inter = jnp.maximum(yy2 - yy1, zero) * jnp.maximum(xx2 - xx1, zero)
            union = barea + car - inter
            iou = jnp.where(union > zero, inter / union, zero)
            suppress = (iou > thr) | selj
            masked_ref[...] = jnp.where(suppress, NEG, masked)
            selo = out_iota == i
            out_ref[0, 0] = jnp.where(selo, by1, out_ref[0, 0])
            out_ref[0, 1] = jnp.where(selo, bx1, out_ref[0, 1])
            out_ref[0, 2] = jnp.where(selo, by2, out_ref[0, 2])
            out_ref[0, 3] = jnp.where(selo, bx2, out_ref[0, 3])

        return carry

    lax.fori_loop(0, N_OUT, nms_body, jnp.int32(0))


@jax.jit
def kernel(rpn_probs, rpn_bbox, anchors):
    batch = rpn_probs.shape[0]
    scores = rpn_probs[:, :, 1]
    scores = jnp.pad(scores, ((0, 0), (0, NPAD - N_ANCHORS)),
                     constant_values=-1.0)
    geom = jnp.concatenate(
        [anchors.transpose(0, 2, 1), rpn_bbox.transpose(0, 2, 1)], axis=1)
    geom = jnp.pad(geom, ((0, 0), (0, 0), (0, NPAD - N_ANCHORS)))

    thrf, thri = pl.pallas_call(
        _threshold_kernel,
        grid=(batch,),
        in_specs=[pl.BlockSpec((1, ROWS, LANES), lambda b: (b, 0, 0))],
        out_specs=[pl.BlockSpec((1, 1, LANES), lambda b: (b, 0, 0)),
                   pl.BlockSpec((1, 1, LANES), lambda b: (b, 0, 0))],
        out_shape=[jax.ShapeDtypeStruct((batch, 1, LANES), jnp.float32),
                   jax.ShapeDtypeStruct((batch, 1, LANES), jnp.int32)],
    )(scores.reshape(batch, ROWS, LANES))

    comp = _compact(scores, geom, thrf.reshape(batch, LANES),
                    thri.reshape(batch, LANES))

    compt = comp[:TRASH].reshape(batch, COMP, ROWW).transpose(0, 2, 1)
    compt = compt[:, :6, :].reshape(batch, 6, CROWS, LANES)

    out = pl.pallas_call(
        _nms_kernel,
        grid=(batch,),
        in_specs=[pl.BlockSpec((1, 6, CROWS, LANES), lambda b: (b, 0, 0, 0))],
        out_specs=pl.BlockSpec((1, 4, 8, LANES), lambda b: (b, 0, 0, 0)),
        out_shape=jax.ShapeDtypeStruct((batch, 4, 8, LANES), jnp.float32),
        scratch_shapes=[pltpu.VMEM((CROWS, LANES), jnp.float32)],
    )(compt)

    out = out.reshape(batch, 4, 8 * LANES)[:, :, :N_OUT]
    return out.transpose(0, 2, 1)


# speedup vs baseline: 1.1403x; 1.1403x over previous
import functools

import jax
import jax.numpy as jnp
import numpy as np
from jax import lax
from jax.experimental import pallas as pl
from jax.experimental.pallas import tpu as pltpu
from jax.experimental.pallas import tpu_sc as plsc

N_ANCHORS = 20000
LANES = 128
ROWS = 160
NPAD = ROWS * LANES
PRE_NMS = 6000
N_OUT = 1000
IOU_THR = 0.7
NEG = np.float32(-1e38)

SC_TILES = 16
SC_PART = NPAD // SC_TILES
SC_CHUNKS = SC_PART // 128
COMP = 6144
CROWS = COMP // LANES
TRASH = 2 * COMP
COMP_ROWS = TRASH + 8
ROWW = 16


def _threshold_kernel(scores_ref, out_ref, outi_ref):
    scores = scores_ref[0]
    bits = lax.bitcast_convert_type(scores, jnp.int32)

    def count_ge(v):
        return jnp.sum((bits >= v).astype(jnp.int32))

    def bs_body(_, state):
        lo, hi = state
        mid = lo + (hi - lo) // 2
        ge = count_ge(mid) >= PRE_NMS
        return (jnp.where(ge, mid, lo), jnp.where(ge, hi, mid))

    lo, _ = lax.fori_loop(
        0, 31, bs_body, (jnp.int32(0), jnp.int32(np.int32(0x7F800000))))
    vstar = lo
    count_gt = jnp.sum((bits > vstar).astype(jnp.int32))
    k_ties = PRE_NMS - count_gt

    row_iota = lax.broadcasted_iota(jnp.int32, (ROWS, LANES), 0)
    col_iota = lax.broadcasted_iota(jnp.int32, (ROWS, LANES), 1)
    idx2d = row_iota * LANES + col_iota
    is_tie = bits == vstar

    def count_tie_lt(i):
        return jnp.sum((is_tie & (idx2d < i)).astype(jnp.int32))

    def bs2_body(_, state):
        lo2, hi2 = state
        mid = lo2 + (hi2 - lo2) // 2
        ge = count_tie_lt(mid) >= k_ties
        return (jnp.where(ge, lo2, mid), jnp.where(ge, mid, hi2))

    _, hi2 = lax.fori_loop(0, 15, bs2_body, (jnp.int32(0), jnp.int32(NPAD)))
    istar = hi2

    lane = lax.broadcasted_iota(jnp.int32, (1, LANES), 1)
    vstar_f = lax.bitcast_convert_type(vstar, jnp.float32)
    out_ref[0] = jnp.where(lane == 0, vstar_f, np.float32(0.0))
    outi_ref[0] = jnp.where(lane == 0, istar, 0)


def _compact_kernel(scores_hbm, geom_hbm, thrf_hbm, thri_hbm, comp_hbm,
                    sco_v, geo_v, thrf_v, thri_v, rows_v, idx_v, cnt_v,
                    counts_sh, allcnt_v, sem):
    b = lax.axis_index("c")
    part = lax.axis_index("s")
    base = part * SC_PART

    pltpu.sync_copy(scores_hbm.at[b, pl.ds(base, SC_PART)], sco_v)
    for ch in range(8):
        pltpu.sync_copy(geom_hbm.at[b, ch, pl.ds(base, SC_PART)],
                        geo_v.at[ch])
    pltpu.sync_copy(thrf_hbm.at[b], thrf_v)
    pltpu.sync_copy(thri_hbm.at[b], thri_v)
    vstar = thrf_v[pl.ds(0, 16)][0]
    istar = thri_v[pl.ds(0, 16)][0]
    lane = lax.broadcasted_iota(jnp.int32, (16,), 0)

    def active_mask(off):
        s16 = sco_v[pl.ds(off, 16)]
        gidx = base + off + lane
        return s16, (s16 > vstar) | ((s16 == vstar) & (gidx < istar))

    acc = jnp.zeros((16,), jnp.float32)
    for i in range(SC_PART // 16):
        _, m = active_mask(i * 16)
        acc = acc + jnp.where(m, jnp.float32(1), jnp.float32(0))
    cnt_v[...] = jnp.full((16,), jnp.sum(acc).astype(jnp.int32), jnp.int32)
    pltpu.sync_copy(cnt_v, counts_sh.at[part])
    plsc.subcore_barrier()
    pltpu.sync_copy(counts_sh, allcnt_v)

    goff = b * COMP
    for p in range(SC_TILES):
        cvec = allcnt_v[p, pl.ds(0, 16)]
        goff = goff + jnp.where(p < part, cvec[0], 0)

    one = np.float32(1.0)
    zero = np.float32(0.0)
    half = np.float32(0.5)
    handles = []
    cc = jnp.int32(0)
    for j in range(SC_CHUNKS):
        for kk in range(8):
            off = j * 128 + kk * 16
            s16, m = active_mask(off)
            sl = pl.ds(off, 16)
            ay1 = geo_v[0, sl]
            ax1 = geo_v[1, sl]
            ay2 = geo_v[2, sl]
            ax2 = geo_v[3, sl]
            dy = geo_v[4, sl] * np.float32(0.1)
            dx = geo_v[5, sl] * np.float32(0.1)
            dh = geo_v[6, sl] * np.float32(0.2)
            dw = geo_v[7, sl] * np.float32(0.2)
            height = ay2 - ay1
            width = ax2 - ax1
            center_y = ay1 + half * height
            center_x = ax1 + half * width
            center_y = center_y + dy * height
            center_x = center_x + dx * width
            height = height * jnp.exp(dh)
            width = width * jnp.exp(dw)
            y1 = center_y - half * height
            x1 = center_x - half * width
            y2 = y1 + height
            x2 = x1 + width
            y1 = jnp.maximum(jnp.minimum(y1, one), zero)
            x1 = jnp.maximum(jnp.minimum(x1, one), zero)
            y2 = jnp.maximum(jnp.minimum(y2, one), zero)
            x2 = jnp.maximum(jnp.minimum(x2, one), zero)
            area = (y2 - y1) * (x2 - x1)

            mf = jnp.where(m, jnp.float32(1), jnp.float32(0))
            excl = (plsc.cumsum(mf) - mf).astype(jnp.int32)
            rowidx = off + lane
            vals = (s16, y1, x1, y2, x2, area)
            for ch, v in enumerate(vals):
                plsc.store_scatter(
                    rows_v, [rowidx, jnp.full((16,), ch, jnp.int32)], v)
            idx_v[j, pl.ds(kk * 16, 16)] = jnp.where(
                m, goff + cc + excl, jnp.int32(TRASH))
            cc = cc + jnp.sum(mf).astype(jnp.int32)
        handles.append(pltpu.async_copy(
            rows_v.at[pl.ds(j * 128, 128)], comp_hbm.at[idx_v.at[j]], sem))
    for h in handles:
        h.wait()


_compact = functools.partial(
    pl.kernel,
    out_type=jax.ShapeDtypeStruct((COMP_ROWS, ROWW), jnp.float32),
    mesh=plsc.VectorSubcoreMesh(core_axis_name="c", subcore_axis_name="s"),
    compiler_params=pltpu.CompilerParams(
        needs_layout_passes=False, use_tc_tiling_on_sc=False),
    scratch_types=[
        pltpu.VMEM((SC_PART,), jnp.float32),
        pltpu.VMEM((8, SC_PART), jnp.float32),
        pltpu.VMEM((LANES,), jnp.float32),
        pltpu.VMEM((LANES,), jnp.int32),
        pltpu.VMEM((SC_PART, ROWW), jnp.float32),
        pltpu.VMEM((SC_CHUNKS, 128), jnp.int32),
        pltpu.VMEM((16,), jnp.int32),
        pltpu.VMEM_SHARED((SC_TILES, 16), jnp.int32),
        pltpu.VMEM((SC_TILES, 16), jnp.int32),
        pltpu.SemaphoreType.DMA,
    ],
)(_compact_kernel)


def _nms_kernel(comp_ref, out_ref):
    batch = comp_ref.shape[0]
    row_iota = lax.broadcasted_iota(jnp.int32, (CROWS, LANES), 0)
    col_iota = lax.broadcasted_iota(jnp.int32, (CROWS, LANES), 1)
    idx2d = row_iota * LANES + col_iota
    zero = np.float32(0.0)
    out_iota = (lax.broadcasted_iota(jnp.int32, (8, LANES), 0) * LANES
                + lax.broadcasted_iota(jnp.int32, (8, LANES), 1))
    thr = np.float32(IOU_THR)
    big = jnp.int32(2 ** 30)
    fz = jnp.float32(0.0)

    masked0 = tuple(
        jnp.where(idx2d < PRE_NMS, comp_ref[b, 0], NEG) for b in range(batch))
    outs0 = tuple(
        jnp.zeros((8, LANES), jnp.float32) for _ in range(4 * batch))

    def nms_body(i, carry):
        maskeds = carry[:batch]
        outs = list(carry[batch:])
        new_masked = []
        selo = out_iota == i
        for b in range(batch):
            masked = maskeds[b]
            m = jnp.max(masked, axis=(0, 1), keepdims=True)
            validb = m >= zero
            sel = masked == m
            j = jnp.min(jnp.where(sel, idx2d, big), axis=(0, 1), keepdims=True)
            selj = idx2d == j
            cy1 = comp_ref[b, 1]
            cx1 = comp_ref[b, 2]
            cy2 = comp_ref[b, 3]
            cx2 = comp_ref[b, 4]
            car = comp_ref[b, 5]
            by1 = jnp.sum(jnp.where(selj, cy1, fz), axis=(0, 1), keepdims=True)
            bx1 = jnp.sum(jnp.where(selj, cx1, fz), axis=(0, 1), keepdims=True)
            by2 = jnp.sum(jnp.where(selj, cy2, fz), axis=(0, 1), keepdims=True)
            bx2 = jnp.sum(jnp.where(selj, cx2, fz), axis=(0, 1), keepdims=True)
            barea = jnp.sum(jnp.where(selj, car, fz),
                            axis=(0, 1), keepdims=True)
            yy1 = jnp.maximum(by1, cy1)
            xx1 = jnp.maximum(bx1, cx1)
            yy2 = jnp.minimum(by2, cy2)
            xx2 = jnp.minimum(bx2, cx2)
            inter = (jnp.maximum(yy2 - yy1, zero)
                     * jnp.maximum(xx2 - xx1, zero))
            union = barea + car - inter
            iou = jnp.where(union > zero, inter / union, zero)
            suppress = ((iou > thr) | selj) & validb
            new_masked.append(jnp.where(suppress, NEG, masked))
            wsel = selo & validb
            outs[4 * b + 0] = jnp.where(wsel, by1, outs[4 * b + 0])
            outs[4 * b + 1] = jnp.where(wsel, bx1, outs[4 * b + 1])
            outs[4 * b + 2] = jnp.where(wsel, by2, outs[4 * b + 2])
            outs[4 * b + 3] = jnp.where(wsel, bx2, outs[4 * b + 3])
        return tuple(new_masked) + tuple(outs)

    fin = lax.fori_loop(0, N_OUT, nms_body, masked0 + outs0)
    for b in range(batch):
        for ch in range(4):
            out_ref[b, ch] = fin[batch + 4 * b + ch]


@jax.jit
def kernel(rpn_probs, rpn_bbox, anchors):
    batch = rpn_probs.shape[0]
    scores = rpn_probs[:, :, 1]
    scores = jnp.pad(scores, ((0, 0), (0, NPAD - N_ANCHORS)),
                     constant_values=-1.0)
    geom = jnp.concatenate(
        [anchors.transpose(0, 2, 1), rpn_bbox.transpose(0, 2, 1)], axis=1)
    geom = jnp.pad(geom, ((0, 0), (0, 0), (0, NPAD - N_ANCHORS)))

    thrf, thri = pl.pallas_call(
        _threshold_kernel,
        grid=(batch,),
        in_specs=[pl.BlockSpec((1, ROWS, LANES), lambda b: (b, 0, 0))],
        out_specs=[pl.BlockSpec((1, 1, LANES), lambda b: (b, 0, 0)),
                   pl.BlockSpec((1, 1, LANES), lambda b: (b, 0, 0))],
        out_shape=[jax.ShapeDtypeStruct((batch, 1, LANES), jnp.float32),
                   jax.ShapeDtypeStruct((batch, 1, LANES), jnp.int32)],
    )(scores.reshape(batch, ROWS, LANES))

    comp = _compact(scores, geom, thrf.reshape(batch, LANES),
                    thri.reshape(batch, LANES))

    compt = comp[:TRASH].reshape(batch, COMP, ROWW).transpose(0, 2, 1)
    compt = compt[:, :6, :].reshape(batch, 6, CROWS, LANES)

    out = pl.pallas_call(
        _nms_kernel,
        out_shape=jax.ShapeDtypeStruct((batch, 4, 8, LANES), jnp.float32),
    )(compt)

    out = out.reshape(batch, 4, 8 * LANES)[:, :, :N_OUT]
    return out.transpose(0, 2, 1)


# speedup vs baseline: 1.1950x; 1.0479x over previous
import functools

import jax
import jax.numpy as jnp
import numpy as np
from jax import lax
from jax.experimental import pallas as pl
from jax.experimental.pallas import tpu as pltpu
from jax.experimental.pallas import tpu_sc as plsc

N_ANCHORS = 20000
LANES = 128
ROWS = 160
NPAD = ROWS * LANES
PRE_NMS = 6000
N_OUT = 1000
IOU_THR = 0.7
NEG = np.float32(-1e38)

SC_TILES = 16
SC_PART = NPAD // SC_TILES
SC_CHUNKS = SC_PART // 128
COMP = 6144
CROWS = COMP // LANES
TRASH = 2 * COMP
COMP_ROWS = TRASH + 8
ROWW = 16


def _threshold_kernel(scores_ref, out_ref, outi_ref):
    scores = scores_ref[0]
    bits = lax.bitcast_convert_type(scores, jnp.int32)

    def count_ge(v):
        return jnp.sum((bits >= v).astype(jnp.int32))

    def bs_body(_, state):
        lo, hi = state
        mid = lo + (hi - lo) // 2
        ge = count_ge(mid) >= PRE_NMS
        return (jnp.where(ge, mid, lo), jnp.where(ge, hi, mid))

    lo, _ = lax.fori_loop(
        0, 31, bs_body, (jnp.int32(0), jnp.int32(np.int32(0x7F800000))))
    vstar = lo
    count_gt = jnp.sum((bits > vstar).astype(jnp.int32))
    k_ties = PRE_NMS - count_gt

    row_iota = lax.broadcasted_iota(jnp.int32, (ROWS, LANES), 0)
    col_iota = lax.broadcasted_iota(jnp.int32, (ROWS, LANES), 1)
    idx2d = row_iota * LANES + col_iota
    is_tie = bits == vstar

    def count_tie_lt(i):
        return jnp.sum((is_tie & (idx2d < i)).astype(jnp.int32))

    def bs2_body(_, state):
        lo2, hi2 = state
        mid = lo2 + (hi2 - lo2) // 2
        ge = count_tie_lt(mid) >= k_ties
        return (jnp.where(ge, lo2, mid), jnp.where(ge, mid, hi2))

    _, hi2 = lax.fori_loop(0, 15, bs2_body, (jnp.int32(0), jnp.int32(NPAD)))
    istar = hi2

    lane = lax.broadcasted_iota(jnp.int32, (1, LANES), 1)
    vstar_f = lax.bitcast_convert_type(vstar, jnp.float32)
    out_ref[0] = jnp.where(lane == 0, vstar_f, np.float32(0.0))
    outi_ref[0] = jnp.where(lane == 0, istar, 0)


def _compact_kernel(scores_hbm, geom_hbm, thrf_hbm, thri_hbm, comp_hbm,
                    sco_v, geo_v, thrf_v, thri_v, rows_v, idx_v, cnt_v,
                    counts_sh, allcnt_v, sem):
    b = lax.axis_index("c")
    part = lax.axis_index("s")
    base = part * SC_PART

    pltpu.sync_copy(scores_hbm.at[b, pl.ds(base, SC_PART)], sco_v)
    for ch in range(8):
        pltpu.sync_copy(geom_hbm.at[b, ch, pl.ds(base, SC_PART)],
                        geo_v.at[ch])
    pltpu.sync_copy(thrf_hbm.at[b], thrf_v)
    pltpu.sync_copy(thri_hbm.at[b], thri_v)
    vstar = thrf_v[pl.ds(0, 16)][0]
    istar = thri_v[pl.ds(0, 16)][0]
    lane = lax.broadcasted_iota(jnp.int32, (16,), 0)

    def active_mask(off):
        s16 = sco_v[pl.ds(off, 16)]
        gidx = base + off + lane
        return s16, (s16 > vstar) | ((s16 == vstar) & (gidx < istar))

    acc = jnp.zeros((16,), jnp.float32)
    for i in range(SC_PART // 16):
        _, m = active_mask(i * 16)
        acc = acc + jnp.where(m, jnp.float32(1), jnp.float32(0))
    cnt_v[...] = jnp.full((16,), jnp.sum(acc).astype(jnp.int32), jnp.int32)
    pltpu.sync_copy(cnt_v, counts_sh.at[part])
    plsc.subcore_barrier()
    pltpu.sync_copy(counts_sh, allcnt_v)

    goff = b * COMP
    for p in range(SC_TILES):
        cvec = allcnt_v[p, pl.ds(0, 16)]
        goff = goff + jnp.where(p < part, cvec[0], 0)

    one = np.float32(1.0)
    zero = np.float32(0.0)
    half = np.float32(0.5)
    handles = []
    cc = jnp.int32(0)
    for j in range(SC_CHUNKS):
        for kk in range(8):
            off = j * 128 + kk * 16
            s16, m = active_mask(off)
            sl = pl.ds(off, 16)
            ay1 = geo_v[0, sl]
            ax1 = geo_v[1, sl]
            ay2 = geo_v[2, sl]
            ax2 = geo_v[3, sl]
            dy = geo_v[4, sl] * np.float32(0.1)
            dx = geo_v[5, sl] * np.float32(0.1)
            dh = geo_v[6, sl] * np.float32(0.2)
            dw = geo_v[7, sl] * np.float32(0.2)
            height = ay2 - ay1
            width = ax2 - ax1
            center_y = ay1 + half * height
            center_x = ax1 + half * width
            center_y = center_y + dy * height
            center_x = center_x + dx * width
            height = height * jnp.exp(dh)
            width = width * jnp.exp(dw)
            y1 = center_y - half * height
            x1 = center_x - half * width
            y2 = y1 + height
            x2 = x1 + width
            y1 = jnp.maximum(jnp.minimum(y1, one), zero)
            x1 = jnp.maximum(jnp.minimum(x1, one), zero)
            y2 = jnp.maximum(jnp.minimum(y2, one), zero)
            x2 = jnp.maximum(jnp.minimum(x2, one), zero)
            area = (y2 - y1) * (x2 - x1)

            mf = jnp.where(m, jnp.float32(1), jnp.float32(0))
            excl = (plsc.cumsum(mf) - mf).astype(jnp.int32)
            rowidx = off + lane
            vals = (s16, y1, x1, y2, x2, area)
            for ch, v in enumerate(vals):
                plsc.store_scatter(
                    rows_v, [rowidx, jnp.full((16,), ch, jnp.int32)], v)
            idx_v[j, pl.ds(kk * 16, 16)] = jnp.where(
                m, goff + cc + excl, jnp.int32(TRASH))
            cc = cc + jnp.sum(mf).astype(jnp.int32)
        handles.append(pltpu.async_copy(
            rows_v.at[pl.ds(j * 128, 128)], comp_hbm.at[idx_v.at[j]], sem))
    for h in handles:
        h.wait()


_compact = functools.partial(
    pl.kernel,
    out_type=jax.ShapeDtypeStruct((COMP_ROWS, ROWW), jnp.float32),
    mesh=plsc.VectorSubcoreMesh(core_axis_name="c", subcore_axis_name="s"),
    compiler_params=pltpu.CompilerParams(
        needs_layout_passes=False, use_tc_tiling_on_sc=False),
    scratch_types=[
        pltpu.VMEM((SC_PART,), jnp.float32),
        pltpu.VMEM((8, SC_PART), jnp.float32),
        pltpu.VMEM((LANES,), jnp.float32),
        pltpu.VMEM((LANES,), jnp.int32),
        pltpu.VMEM((SC_PART, ROWW), jnp.float32),
        pltpu.VMEM((SC_CHUNKS, 128), jnp.int32),
        pltpu.VMEM((16,), jnp.int32),
        pltpu.VMEM_SHARED((SC_TILES, 16), jnp.int32),
        pltpu.VMEM((SC_TILES, 16), jnp.int32),
        pltpu.SemaphoreType.DMA,
    ],
)(_compact_kernel)


def _nms_kernel(comp_ref, raw_ref, out_ref):
    batch = comp_ref.shape[0]
    row_iota = lax.broadcasted_iota(jnp.int32, (CROWS, LANES), 0)
    col_iota = lax.broadcasted_iota(jnp.int32, (CROWS, LANES), 1)
    idx2d = row_iota * LANES + col_iota
    zero = np.float32(0.0)
    out_iota = (lax.broadcasted_iota(jnp.int32, (8, LANES), 0) * LANES
                + lax.broadcasted_iota(jnp.int32, (8, LANES), 1))
    thr = np.float32(IOU_THR)
    big = jnp.int32(2 ** 30)
    fz = jnp.float32(0.0)

    masked0 = tuple(
        jnp.where(idx2d < PRE_NMS, comp_ref[b, 0], NEG) for b in range(batch))
    outs0 = tuple(
        jnp.zeros((8, LANES), jnp.float32) for _ in range(4 * batch))

    def nms_body(i, carry):
        maskeds = carry[:batch]
        outs = list(carry[batch:])
        new_masked = []
        selo = out_iota == i
        for b in range(batch):
            masked = maskeds[b]
            m = jnp.max(masked, axis=(0, 1), keepdims=True)
            validb = m >= zero
            sel = masked == m
            j = jnp.min(jnp.where(sel, idx2d, big))
            selj = idx2d == j
            cy1 = comp_ref[b, 1]
            cx1 = comp_ref[b, 2]
            cy2 = comp_ref[b, 3]
            cx2 = comp_ref[b, 4]
            car = comp_ref[b, 5]
            row = raw_ref[b, pl.ds(j, 1), :]
            by1 = row[0:1, 1:2]
            bx1 = row[0:1, 2:3]
            by2 = row[0:1, 3:4]
            bx2 = row[0:1, 4:5]
            barea = row[0:1, 5:6]
            yy1 = jnp.maximum(by1, cy1)
            xx1 = jnp.maximum(bx1, cx1)
            yy2 = jnp.minimum(by2, cy2)
            xx2 = jnp.minimum(bx2, cx2)
            inter = (jnp.maximum(yy2 - yy1, zero)
                     * jnp.maximum(xx2 - xx1, zero))
            union = barea + car - inter
            iou = jnp.where(union > zero, inter / union, zero)
            suppress = ((iou > thr) | selj) & validb
            new_masked.append(jnp.where(suppress, NEG, masked))
            wsel = selo & validb
            outs[4 * b + 0] = jnp.where(wsel, by1, outs[4 * b + 0])
            outs[4 * b + 1] = jnp.where(wsel, bx1, outs[4 * b + 1])
            outs[4 * b + 2] = jnp.where(wsel, by2, outs[4 * b + 2])
            outs[4 * b + 3] = jnp.where(wsel, bx2, outs[4 * b + 3])
        return tuple(new_masked) + tuple(outs)

    fin = lax.fori_loop(0, N_OUT, nms_body, masked0 + outs0)
    for b in range(batch):
        for ch in range(4):
            out_ref[b, ch] = fin[batch + 4 * b + ch]


@jax.jit
def kernel(rpn_probs, rpn_bbox, anchors):
    batch = rpn_probs.shape[0]
    scores = rpn_probs[:, :, 1]
    scores = jnp.pad(scores, ((0, 0), (0, NPAD - N_ANCHORS)),
                     constant_values=-1.0)
    geom = jnp.concatenate(
        [anchors.transpose(0, 2, 1), rpn_bbox.transpose(0, 2, 1)], axis=1)
    geom = jnp.pad(geom, ((0, 0), (0, 0), (0, NPAD - N_ANCHORS)))

    thrf, thri = pl.pallas_call(
        _threshold_kernel,
        grid=(batch,),
        in_specs=[pl.BlockSpec((1, ROWS, LANES), lambda b: (b, 0, 0))],
        out_specs=[pl.BlockSpec((1, 1, LANES), lambda b: (b, 0, 0)),
                   pl.BlockSpec((1, 1, LANES), lambda b: (b, 0, 0))],
        out_shape=[jax.ShapeDtypeStruct((batch, 1, LANES), jnp.float32),
                   jax.ShapeDtypeStruct((batch, 1, LANES), jnp.int32)],
    )(scores.reshape(batch, ROWS, LANES))

    comp = _compact(scores, geom, thrf.reshape(batch, LANES),
                    thri.reshape(batch, LANES))

    raw = comp[:TRASH].reshape(batch, COMP, ROWW)
    compt = raw.transpose(0, 2, 1)[:, :6, :].reshape(batch, 6, CROWS, LANES)

    out = pl.pallas_call(
        _nms_kernel,
        out_shape=jax.ShapeDtypeStruct((batch, 4, 8, LANES), jnp.float32),
    )(compt, raw)

    out = out.reshape(batch, 4, 8 * LANES)[:, :, :N_OUT]
    return out.transpose(0, 2, 1)


# speedup vs baseline: 1.7295x; 1.4473x over previous
import functools

import jax
import jax.numpy as jnp
import numpy as np
from jax import lax
from jax.experimental import pallas as pl
from jax.experimental.pallas import tpu as pltpu
from jax.experimental.pallas import tpu_sc as plsc

N_ANCHORS = 20000
LANES = 128
ROWS = 160
NPAD = ROWS * LANES
PRE_NMS = 6000
N_OUT = 1000
IOU_THR = 0.7
NEG = np.float32(-1e38)

SC_TILES = 16
SC_PART = NPAD // SC_TILES
SC_CHUNKS = SC_PART // 128
COMP = 6144
CROWS = COMP // LANES
TRASH = 2 * COMP
COMP_ROWS = TRASH + 8
ROWW = 16


def _threshold_kernel(scores_ref, out_ref, outi_ref):
    scores = scores_ref[0]
    bits = lax.bitcast_convert_type(scores, jnp.int32)

    def count_ge(v):
        return jnp.sum((bits >= v).astype(jnp.int32))

    def bs_body(_, state):
        lo, hi = state
        mid = lo + (hi - lo) // 2
        ge = count_ge(mid) >= PRE_NMS
        return (jnp.where(ge, mid, lo), jnp.where(ge, hi, mid))

    lo, _ = lax.fori_loop(
        0, 31, bs_body, (jnp.int32(0), jnp.int32(np.int32(0x7F800000))))
    vstar = lo
    count_gt = jnp.sum((bits > vstar).astype(jnp.int32))
    k_ties = PRE_NMS - count_gt

    row_iota = lax.broadcasted_iota(jnp.int32, (ROWS, LANES), 0)
    col_iota = lax.broadcasted_iota(jnp.int32, (ROWS, LANES), 1)
    idx2d = row_iota * LANES + col_iota
    is_tie = bits == vstar

    def count_tie_lt(i):
        return jnp.sum((is_tie & (idx2d < i)).astype(jnp.int32))

    def bs2_body(_, state):
        lo2, hi2 = state
        mid = lo2 + (hi2 - lo2) // 2
        ge = count_tie_lt(mid) >= k_ties
        return (jnp.where(ge, lo2, mid), jnp.where(ge, mid, hi2))

    _, hi2 = lax.fori_loop(0, 15, bs2_body, (jnp.int32(0), jnp.int32(NPAD)))
    istar = hi2

    lane = lax.broadcasted_iota(jnp.int32, (1, LANES), 1)
    vstar_f = lax.bitcast_convert_type(vstar, jnp.float32)
    out_ref[0] = jnp.where(lane == 0, vstar_f, np.float32(0.0))
    outi_ref[0] = jnp.where(lane == 0, istar, 0)


def _compact_kernel(scores_hbm, geom_hbm, thrf_hbm, thri_hbm, comp_hbm,
                    sco_v, geo_v, thrf_v, thri_v, rows_v, idx_v, cnt_v,
                    counts_sh, allcnt_v, sem):
    b = lax.axis_index("c")
    part = lax.axis_index("s")
    base = part * SC_PART

    pltpu.sync_copy(scores_hbm.at[b, pl.ds(base, SC_PART)], sco_v)
    for ch in range(8):
        pltpu.sync_copy(geom_hbm.at[b, ch, pl.ds(base, SC_PART)],
                        geo_v.at[ch])
    pltpu.sync_copy(thrf_hbm.at[b], thrf_v)
    pltpu.sync_copy(thri_hbm.at[b], thri_v)
    vstar = thrf_v[pl.ds(0, 16)][0]
    istar = thri_v[pl.ds(0, 16)][0]
    lane = lax.broadcasted_iota(jnp.int32, (16,), 0)

    def active_mask(off):
        s16 = sco_v[pl.ds(off, 16)]
        gidx = base + off + lane
        return s16, (s16 > vstar) | ((s16 == vstar) & (gidx < istar))

    acc = jnp.zeros((16,), jnp.float32)
    for i in range(SC_PART // 16):
        _, m = active_mask(i * 16)
        acc = acc + jnp.where(m, jnp.float32(1), jnp.float32(0))
    cnt_v[...] = jnp.full((16,), jnp.sum(acc).astype(jnp.int32), jnp.int32)
    pltpu.sync_copy(cnt_v, counts_sh.at[part])
    plsc.subcore_barrier()
    pltpu.sync_copy(counts_sh, allcnt_v)

    goff = b * COMP
    for p in range(SC_TILES):
        cvec = allcnt_v[p, pl.ds(0, 16)]
        goff = goff + jnp.where(p < part, cvec[0], 0)

    one = np.float32(1.0)
    zero = np.float32(0.0)
    half = np.float32(0.5)
    handles = []
    cc = jnp.int32(0)
    for j in range(SC_CHUNKS):
        for kk in range(8):
            off = j * 128 + kk * 16
            s16, m = active_mask(off)
            sl = pl.ds(off, 16)
            ay1 = geo_v[0, sl]
            ax1 = geo_v[1, sl]
            ay2 = geo_v[2, sl]
            ax2 = geo_v[3, sl]
            dy = geo_v[4, sl] * np.float32(0.1)
            dx = geo_v[5, sl] * np.float32(0.1)
            dh = geo_v[6, sl] * np.float32(0.2)
            dw = geo_v[7, sl] * np.float32(0.2)
            height = ay2 - ay1
            width = ax2 - ax1
            center_y = ay1 + half * height
            center_x = ax1 + half * width
            center_y = center_y + dy * height
            center_x = center_x + dx * width
            height = height * jnp.exp(dh)
            width = width * jnp.exp(dw)
            y1 = center_y - half * height
            x1 = center_x - half * width
            y2 = y1 + height
            x2 = x1 + width
            y1 = jnp.maximum(jnp.minimum(y1, one), zero)
            x1 = jnp.maximum(jnp.minimum(x1, one), zero)
            y2 = jnp.maximum(jnp.minimum(y2, one), zero)
            x2 = jnp.maximum(jnp.minimum(x2, one), zero)
            area = (y2 - y1) * (x2 - x1)

            mf = jnp.where(m, jnp.float32(1), jnp.float32(0))
            excl = (plsc.cumsum(mf) - mf).astype(jnp.int32)
            rowidx = off + lane
            vals = (s16, y1, x1, y2, x2, area)
            for ch, v in enumerate(vals):
                plsc.store_scatter(
                    rows_v, [rowidx, jnp.full((16,), ch, jnp.int32)], v)
            idx_v[j, pl.ds(kk * 16, 16)] = jnp.where(
                m, goff + cc + excl, jnp.int32(TRASH))
            cc = cc + jnp.sum(mf).astype(jnp.int32)
        handles.append(pltpu.async_copy(
            rows_v.at[pl.ds(j * 128, 128)], comp_hbm.at[idx_v.at[j]], sem))
    for h in handles:
        h.wait()


_compact = functools.partial(
    pl.kernel,
    out_type=jax.ShapeDtypeStruct((COMP_ROWS, ROWW), jnp.float32),
    mesh=plsc.VectorSubcoreMesh(core_axis_name="c", subcore_axis_name="s"),
    compiler_params=pltpu.CompilerParams(
        needs_layout_passes=False, use_tc_tiling_on_sc=False),
    scratch_types=[
        pltpu.VMEM((SC_PART,), jnp.float32),
        pltpu.VMEM((8, SC_PART), jnp.float32),
        pltpu.VMEM((LANES,), jnp.float32),
        pltpu.VMEM((LANES,), jnp.int32),
        pltpu.VMEM((SC_PART, ROWW), jnp.float32),
        pltpu.VMEM((SC_CHUNKS, 128), jnp.int32),
        pltpu.VMEM((16,), jnp.int32),
        pltpu.VMEM_SHARED((SC_TILES, 16), jnp.int32),
        pltpu.VMEM((SC_TILES, 16), jnp.int32),
        pltpu.SemaphoreType.DMA,
    ],
)(_compact_kernel)


def _nms_kernel(comp_ref, raw_ref, out_ref):
    batch = comp_ref.shape[0]
    row_iota = lax.broadcasted_iota(jnp.int32, (CROWS, LANES), 0)
    col_iota = lax.broadcasted_iota(jnp.int32, (CROWS, LANES), 1)
    idx2d = row_iota * LANES + col_iota
    zero = np.float32(0.0)
    out_iota = (lax.broadcasted_iota(jnp.int32, (8, LANES), 0) * LANES
                + lax.broadcasted_iota(jnp.int32, (8, LANES), 1))
    thr = np.float32(IOU_THR)
    big = jnp.int32(2 ** 30)
    fz = jnp.float32(0.0)

    masked0 = tuple(
        jnp.where(idx2d < PRE_NMS, comp_ref[b, 0], NEG) for b in range(batch))
    outs0 = tuple(
        jnp.zeros((8, LANES), jnp.float32) for _ in range(4 * batch))

    def nms_body(i, carry):
        maskeds = carry[:batch]
        outs = list(carry[batch:])
        selo = out_iota == i
        ms = [jnp.max(maskeds[b], axis=(0, 1), keepdims=True)
              for b in range(batch)]
        valids = [ms[b] >= zero for b in range(batch)]
        sels = [maskeds[b] == ms[b] for b in range(batch)]
        js = [jnp.min(jnp.where(sels[b], idx2d, big)) for b in range(batch)]
        rows = [raw_ref[b, pl.ds(js[b], 1), :] for b in range(batch)]
        new_masked = []
        for b in range(batch):
            row = rows[b]
            by1 = row[0:1, 1:2]
            bx1 = row[0:1, 2:3]
            by2 = row[0:1, 3:4]
            bx2 = row[0:1, 4:5]
            barea = row[0:1, 5:6]
            selj = idx2d == js[b]
            cy1 = comp_ref[b, 1]
            cx1 = comp_ref[b, 2]
            cy2 = comp_ref[b, 3]
            cx2 = comp_ref[b, 4]
            car = comp_ref[b, 5]
            yy1 = jnp.maximum(by1, cy1)
            xx1 = jnp.maximum(bx1, cx1)
            yy2 = jnp.minimum(by2, cy2)
            xx2 = jnp.minimum(bx2, cx2)
            inter = (jnp.maximum(yy2 - yy1, zero)
                     * jnp.maximum(xx2 - xx1, zero))
            union = barea + car - inter
            iou = jnp.where(union > zero, inter / union, zero)
            suppress = ((iou > thr) | selj) & valids[b]
            new_masked.append(jnp.where(suppress, NEG, maskeds[b]))
            wsel = selo & valids[b]
            outs[4 * b + 0] = jnp.where(wsel, by1, outs[4 * b + 0])
            outs[4 * b + 1] = jnp.where(wsel, bx1, outs[4 * b + 1])
            outs[4 * b + 2] = jnp.where(wsel, by2, outs[4 * b + 2])
            outs[4 * b + 3] = jnp.where(wsel, bx2, outs[4 * b + 3])
        return tuple(new_masked) + tuple(outs)

    fin = lax.fori_loop(0, N_OUT, nms_body, masked0 + outs0)
    for b in range(batch):
        for ch in range(4):
            out_ref[b, ch] = fin[batch + 4 * b + ch]


@jax.jit
def kernel(rpn_probs, rpn_bbox, anchors):
    batch = rpn_probs.shape[0]
    scores = rpn_probs[:, :, 1]
    scores = jnp.pad(scores, ((0, 0), (0, NPAD - N_ANCHORS)),
                     constant_values=-1.0)
    geom = jnp.concatenate(
        [anchors.transpose(0, 2, 1), rpn_bbox.transpose(0, 2, 1)], axis=1)
    geom = jnp.pad(geom, ((0, 0), (0, 0), (0, NPAD - N_ANCHORS)))

    thrf, thri = pl.pallas_call(
        _threshold_kernel,
        grid=(batch,),
        in_specs=[pl.BlockSpec((1, ROWS, LANES), lambda b: (b, 0, 0))],
        out_specs=[pl.BlockSpec((1, 1, LANES), lambda b: (b, 0, 0)),
                   pl.BlockSpec((1, 1, LANES), lambda b: (b, 0, 0))],
        out_shape=[jax.ShapeDtypeStruct((batch, 1, LANES), jnp.float32),
                   jax.ShapeDtypeStruct((batch, 1, LANES), jnp.int32)],
    )(scores.reshape(batch, ROWS, LANES))

    comp = _compact(scores, geom, thrf.reshape(batch, LANES),
                    thri.reshape(batch, LANES))

    raw = comp[:TRASH].reshape(batch, COMP, ROWW)
    compt = raw.transpose(0, 2, 1)[:, :6, :].reshape(batch, 6, CROWS, LANES)

    out = pl.pallas_call(
        _nms_kernel,
        out_shape=jax.ShapeDtypeStruct((batch, 4, 8, LANES), jnp.float32),
    )(compt, raw)

    out = out.reshape(batch, 4, 8 * LANES)[:, :, :N_OUT]
    return out.transpose(0, 2, 1)


# speedup vs baseline: 2.0535x; 1.1873x over previous
import functools

import jax
import jax.numpy as jnp
import numpy as np
from jax import lax
from jax.experimental import pallas as pl
from jax.experimental.pallas import tpu as pltpu
from jax.experimental.pallas import tpu_sc as plsc

N_ANCHORS = 20000
LANES = 128
ROWS = 160
NPAD = ROWS * LANES
PRE_NMS = 6000
N_OUT = 1000
IOU_THR = 0.7
NEG = np.float32(-1e38)

SC_TILES = 16
SC_PART = NPAD // SC_TILES
SC_CHUNKS = SC_PART // 128
COMP = 6144
CROWS = COMP // LANES
TRASH = 2 * COMP
COMP_ROWS = TRASH + 8
ROWW = 16


def _threshold_kernel(scores_ref, out_ref, outi_ref):
    scores = scores_ref[0]
    bits = lax.bitcast_convert_type(scores, jnp.int32)

    def count_ge(v):
        return jnp.sum((bits >= v).astype(jnp.int32))

    def bs_body(_, state):
        lo, hi = state
        mid = lo + (hi - lo) // 2
        ge = count_ge(mid) >= PRE_NMS
        return (jnp.where(ge, mid, lo), jnp.where(ge, hi, mid))

    lo, _ = lax.fori_loop(
        0, 31, bs_body, (jnp.int32(0), jnp.int32(np.int32(0x7F800000))))
    vstar = lo
    count_gt = jnp.sum((bits > vstar).astype(jnp.int32))
    k_ties = PRE_NMS - count_gt

    row_iota = lax.broadcasted_iota(jnp.int32, (ROWS, LANES), 0)
    col_iota = lax.broadcasted_iota(jnp.int32, (ROWS, LANES), 1)
    idx2d = row_iota * LANES + col_iota
    is_tie = bits == vstar

    def count_tie_lt(i):
        return jnp.sum((is_tie & (idx2d < i)).astype(jnp.int32))

    def bs2_body(_, state):
        lo2, hi2 = state
        mid = lo2 + (hi2 - lo2) // 2
        ge = count_tie_lt(mid) >= k_ties
        return (jnp.where(ge, lo2, mid), jnp.where(ge, mid, hi2))

    _, hi2 = lax.fori_loop(0, 15, bs2_body, (jnp.int32(0), jnp.int32(NPAD)))
    istar = hi2

    lane = lax.broadcasted_iota(jnp.int32, (1, LANES), 1)
    vstar_f = lax.bitcast_convert_type(vstar, jnp.float32)
    out_ref[0] = jnp.where(lane == 0, vstar_f, np.float32(0.0))
    outi_ref[0] = jnp.where(lane == 0, istar, 0)


def _compact_kernel(scores_hbm, geom_hbm, thrf_hbm, thri_hbm, comp_hbm,
                    sco_v, geo_v, thrf_v, thri_v, rows_v, idx_v, cnt_v,
                    counts_sh, allcnt_v, sem):
    b = lax.axis_index("c")
    part = lax.axis_index("s")
    base = part * SC_PART

    pltpu.sync_copy(scores_hbm.at[b, pl.ds(base, SC_PART)], sco_v)
    for ch in range(8):
        pltpu.sync_copy(geom_hbm.at[b, ch, pl.ds(base, SC_PART)],
                        geo_v.at[ch])
    pltpu.sync_copy(thrf_hbm.at[b], thrf_v)
    pltpu.sync_copy(thri_hbm.at[b], thri_v)
    vstar = thrf_v[pl.ds(0, 16)][0]
    istar = thri_v[pl.ds(0, 16)][0]
    lane = lax.broadcasted_iota(jnp.int32, (16,), 0)

    def active_mask(off):
        s16 = sco_v[pl.ds(off, 16)]
        gidx = base + off + lane
        return s16, (s16 > vstar) | ((s16 == vstar) & (gidx < istar))

    acc = jnp.zeros((16,), jnp.float32)
    for i in range(SC_PART // 16):
        _, m = active_mask(i * 16)
        acc = acc + jnp.where(m, jnp.float32(1), jnp.float32(0))
    cnt_v[...] = jnp.full((16,), jnp.sum(acc).astype(jnp.int32), jnp.int32)
    pltpu.sync_copy(cnt_v, counts_sh.at[part])
    plsc.subcore_barrier()
    pltpu.sync_copy(counts_sh, allcnt_v)

    goff = b * COMP
    for p in range(SC_TILES):
        cvec = allcnt_v[p, pl.ds(0, 16)]
        goff = goff + jnp.where(p < part, cvec[0], 0)

    one = np.float32(1.0)
    zero = np.float32(0.0)
    half = np.float32(0.5)
    handles = []
    cc = jnp.int32(0)
    for j in range(SC_CHUNKS):
        for kk in range(8):
            off = j * 128 + kk * 16
            s16, m = active_mask(off)
            sl = pl.ds(off, 16)
            ay1 = geo_v[0, sl]
            ax1 = geo_v[1, sl]
            ay2 = geo_v[2, sl]
            ax2 = geo_v[3, sl]
            dy = geo_v[4, sl] * np.float32(0.1)
            dx = geo_v[5, sl] * np.float32(0.1)
            dh = geo_v[6, sl] * np.float32(0.2)
            dw = geo_v[7, sl] * np.float32(0.2)
            height = ay2 - ay1
            width = ax2 - ax1
            center_y = ay1 + half * height
            center_x = ax1 + half * width
            center_y = center_y + dy * height
            center_x = center_x + dx * width
            height = height * jnp.exp(dh)
            width = width * jnp.exp(dw)
            y1 = center_y - half * height
            x1 = center_x - half * width
            y2 = y1 + height
            x2 = x1 + width
            y1 = jnp.maximum(jnp.minimum(y1, one), zero)
            x1 = jnp.maximum(jnp.minimum(x1, one), zero)
            y2 = jnp.maximum(jnp.minimum(y2, one), zero)
            x2 = jnp.maximum(jnp.minimum(x2, one), zero)
            area = (y2 - y1) * (x2 - x1)

            mf = jnp.where(m, jnp.float32(1), jnp.float32(0))
            excl = (plsc.cumsum(mf) - mf).astype(jnp.int32)
            rowidx = off + lane
            vals = (s16, y1, x1, y2, x2, area)
            for ch, v in enumerate(vals):
                plsc.store_scatter(
                    rows_v, [rowidx, jnp.full((16,), ch, jnp.int32)], v)
            idx_v[j, pl.ds(kk * 16, 16)] = jnp.where(
                m, goff + cc + excl, jnp.int32(TRASH))
            cc = cc + jnp.sum(mf).astype(jnp.int32)
        handles.append(pltpu.async_copy(
            rows_v.at[pl.ds(j * 128, 128)], comp_hbm.at[idx_v.at[j]], sem))
    for h in handles:
        h.wait()


_compact = functools.partial(
    pl.kernel,
    out_type=jax.ShapeDtypeStruct((COMP_ROWS, ROWW), jnp.float32),
    mesh=plsc.VectorSubcoreMesh(core_axis_name="c", subcore_axis_name="s"),
    compiler_params=pltpu.CompilerParams(
        needs_layout_passes=False, use_tc_tiling_on_sc=False),
    scratch_types=[
        pltpu.VMEM((SC_PART,), jnp.float32),
        pltpu.VMEM((8, SC_PART), jnp.float32),
        pltpu.VMEM((LANES,), jnp.float32),
        pltpu.VMEM((LANES,), jnp.int32),
        pltpu.VMEM((SC_PART, ROWW), jnp.float32),
        pltpu.VMEM((SC_CHUNKS, 128), jnp.int32),
        pltpu.VMEM((16,), jnp.int32),
        pltpu.VMEM_SHARED((SC_TILES, 16), jnp.int32),
        pltpu.VMEM((SC_TILES, 16), jnp.int32),
        pltpu.SemaphoreType.DMA,
    ],
)(_compact_kernel)


def _nms_kernel(comp_ref, raw_ref, out_ref):
    batch = comp_ref.shape[0]
    row_iota = lax.broadcasted_iota(jnp.int32, (CROWS, LANES), 0)
    col_iota = lax.broadcasted_iota(jnp.int32, (CROWS, LANES), 1)
    idx2d = row_iota * LANES + col_iota
    idx2df = idx2d.astype(jnp.float32)
    zero = np.float32(0.0)
    out_iota = (lax.broadcasted_iota(jnp.int32, (8, LANES), 0) * LANES
                + lax.broadcasted_iota(jnp.int32, (8, LANES), 1))
    thr = np.float32(IOU_THR)
    big = jnp.int32(2 ** 30)
    fz = jnp.float32(0.0)

    masked0 = tuple(
        jnp.where(idx2d < PRE_NMS, comp_ref[b, 0], NEG) for b in range(batch))
    outs0 = tuple(
        jnp.zeros((8, LANES), jnp.float32) for _ in range(4 * batch))

    def nms_body(i, carry):
        maskeds = carry[:batch]
        outs = list(carry[batch:])
        selo = out_iota == i
        mbs = []
        valids8 = []
        for b in range(batch):
            v8 = maskeds[b].reshape(6, 8, LANES)
            v8 = jnp.max(v8, axis=0)
            mrow = jnp.max(v8, axis=1, keepdims=True)
            mb = jnp.broadcast_to(mrow, (8, LANES))
            mb = jnp.maximum(mb, pltpu.roll(mb, 4, 0))
            mb = jnp.maximum(mb, pltpu.roll(mb, 2, 0))
            mb = jnp.maximum(mb, pltpu.roll(mb, 1, 0))
            valids8.append(mb >= zero)
            mbs.append(jnp.broadcast_to(
                mb.reshape(1, 8, LANES), (6, 8, LANES)).reshape(CROWS, LANES))
        valids = [jnp.broadcast_to(
            valids8[b].reshape(1, 8, LANES),
            (6, 8, LANES)).reshape(CROWS, LANES) for b in range(batch)]
        sels = [maskeds[b] == mbs[b] for b in range(batch)]
        js = [jnp.min(jnp.where(sels[b], idx2df, jnp.float32(1e9)))
              .astype(jnp.int32) for b in range(batch)]
        rows = [raw_ref[b, pl.ds(js[b], 1), :] for b in range(batch)]
        new_masked = []
        for b in range(batch):
            row = rows[b]
            by1 = row[0:1, 1:2]
            bx1 = row[0:1, 2:3]
            by2 = row[0:1, 3:4]
            bx2 = row[0:1, 4:5]
            barea = row[0:1, 5:6]
            selj = idx2d == js[b]
            cy1 = comp_ref[b, 1]
            cx1 = comp_ref[b, 2]
            cy2 = comp_ref[b, 3]
            cx2 = comp_ref[b, 4]
            car = comp_ref[b, 5]
            yy1 = jnp.maximum(by1, cy1)
            xx1 = jnp.maximum(bx1, cx1)
            yy2 = jnp.minimum(by2, cy2)
            xx2 = jnp.minimum(bx2, cx2)
            inter = (jnp.maximum(yy2 - yy1, zero)
                     * jnp.maximum(xx2 - xx1, zero))
            union = barea + car - inter
            iou = jnp.where(union > zero, inter / union, zero)
            suppress = ((iou > thr) | selj) & valids[b]
            new_masked.append(jnp.where(suppress, NEG, maskeds[b]))
            wsel = selo & valids8[b]
            outs[4 * b + 0] = jnp.where(wsel, by1, outs[4 * b + 0])
            outs[4 * b + 1] = jnp.where(wsel, bx1, outs[4 * b + 1])
            outs[4 * b + 2] = jnp.where(wsel, by2, outs[4 * b + 2])
            outs[4 * b + 3] = jnp.where(wsel, bx2, outs[4 * b + 3])
        return tuple(new_masked) + tuple(outs)

    fin = lax.fori_loop(0, N_OUT, nms_body, masked0 + outs0)
    for b in range(batch):
        for ch in range(4):
            out_ref[b, ch] = fin[batch + 4 * b + ch]


@jax.jit
def kernel(rpn_probs, rpn_bbox, anchors):
    batch = rpn_probs.shape[0]
    scores = rpn_probs[:, :, 1]
    scores = jnp.pad(scores, ((0, 0), (0, NPAD - N_ANCHORS)),
                     constant_values=-1.0)
    geom = jnp.concatenate(
        [anchors.transpose(0, 2, 1), rpn_bbox.transpose(0, 2, 1)], axis=1)
    geom = jnp.pad(geom, ((0, 0), (0, 0), (0, NPAD - N_ANCHORS)))

    thrf, thri = pl.pallas_call(
        _threshold_kernel,
        grid=(batch,),
        in_specs=[pl.BlockSpec((1, ROWS, LANES), lambda b: (b, 0, 0))],
        out_specs=[pl.BlockSpec((1, 1, LANES), lambda b: (b, 0, 0)),
                   pl.BlockSpec((1, 1, LANES), lambda b: (b, 0, 0))],
        out_shape=[jax.ShapeDtypeStruct((batch, 1, LANES), jnp.float32),
                   jax.ShapeDtypeStruct((batch, 1, LANES), jnp.int32)],
    )(scores.reshape(batch, ROWS, LANES))

    comp = _compact(scores, geom, thrf.reshape(batch, LANES),
                    thri.reshape(batch, LANES))

    raw = comp[:TRASH].reshape(batch, COMP, ROWW)
    compt = raw.transpose(0, 2, 1)[:, :6, :].reshape(batch, 6, CROWS, LANES)

    out = pl.pallas_call(
        _nms_kernel,
        out_shape=jax.ShapeDtypeStruct((batch, 4, 8, LANES), jnp.float32),
    )(compt, raw)

    out = out.reshape(batch, 4, 8 * LANES)[:, :, :N_OUT]
    return out.transpose(0, 2, 1)
